# baseline (speedup 1.0000x reference)
import functools

import jax
import jax.numpy as jnp
from jax import lax
from jax.experimental import pallas as pl
from jax.experimental.pallas import tpu as pltpu
from jax.experimental.pallas import tpu_sc as plsc

_D = 32
_NC = 2
_NS = 16
_NW = _NC * _NS
_L = 16


@functools.lru_cache(maxsize=None)
def _make_gather(B, C):
    b_per_w = B // _NW
    CW = C // 4
    n_chunks = b_per_w // C
    assert n_chunks * C == b_per_w and n_chunks % 2 == 0, (B, C)
    assert C % 128 == 0
    n_groups = C // _L
    mesh = plsc.VectorSubcoreMesh(core_axis_name="c", subcore_axis_name="s")

    @functools.partial(
        pl.kernel,
        mesh=mesh,
        out_type=jax.ShapeDtypeStruct((B // 4, 128), jnp.float32),
        scratch_types=[
            pltpu.VMEM((C,), jnp.int32),
            pltpu.VMEM((C,), jnp.int32),
            pltpu.VMEM((C,), jnp.int32),
            pltpu.VMEM((C,), jnp.int32),
            pltpu.VMEM((2, C, 128), jnp.float32),
            pltpu.VMEM((2, CW, 128), jnp.float32),
            pltpu.SemaphoreType.DMA((2,)),
            pltpu.SemaphoreType.DMA((2,)),
            pltpu.SemaphoreType.DMA((2,)),
        ],
        compiler_params=pltpu.CompilerParams(
            use_tc_tiling_on_sc=True, needs_layout_passes=False),
    )
    def k(table_hbm, idx_hbm, out_hbm, idx0_v, idx1_v, cidx0_v, cidx1_v,
          rows_v, out_v, sem_i, sem_g, sem_s):
        idx_vs = (idx0_v, idx1_v)
        cidx_vs = (cidx0_v, cidx1_v)
        wid = lax.axis_index("s") * _NC + lax.axis_index("c")
        base = wid * b_per_w

        def idx_cp(i, b):
            return pltpu.make_async_copy(
                idx_hbm.at[pl.ds(base + i * C, C)], idx_vs[b], sem_i.at[b])

        def gather_cp(b):
            return pltpu.make_async_copy(
                table_hbm.at[cidx_vs[b]], rows_v.at[b], sem_g.at[b])

        def store_cp(i, b):
            q0 = pl.multiple_of((base + i * C) // 4, CW)
            return pltpu.make_async_copy(
                out_v.at[b], out_hbm.at[pl.ds(q0, CW)], sem_s.at[b])

        def prep_cidx(b):
            def grp(jj, carry):
                sl = pl.ds(jj * _L, _L)
                cidx_vs[b][sl] = lax.shift_right_logical(
                    idx_vs[b][sl], jnp.int32(2))
                return carry
            lax.fori_loop(0, n_groups, grp, 0)

        def extract(b):
            iot = lax.iota(jnp.int32, _L)
            qoff = lax.shift_right_logical(iot, jnp.int32(2))
            colu = lax.shift_left(jnp.bitwise_and(iot, jnp.int32(3)),
                                  jnp.int32(5))

            def grp(jj, carry):
                j0 = jj * _L
                rowj = j0 + iot
                rowq = (jj * 4) + qoff
                colbase = lax.shift_left(
                    jnp.bitwise_and(idx_vs[b][pl.ds(j0, _L)], jnp.int32(3)),
                    jnp.int32(5))
                for c in range(_D):
                    vals = plsc.load_gather(
                        rows_v.at[b], [rowj, colbase + jnp.int32(c)])
                    plsc.store_scatter(
                        out_v.at[b], [rowq, colu + jnp.int32(c)], vals)
                return carry
            lax.fori_loop(0, n_groups, grp, 0)

        idx_cp(0, 0).start()
        idx_cp(1, 1).start()
        idx_cp(0, 0).wait()
        prep_cidx(0)
        gather_cp(0).start()

        def chunk_step(i, b):
            gather_cp(b).wait()

            @pl.when(i + 1 < n_chunks)
            def _():
                idx_cp(i + 1, b ^ 1).wait()
                prep_cidx(b ^ 1)
                gather_cp(b ^ 1).start()

            @pl.when(i >= 2)
            def _():
                store_cp(i - 2, b).wait()

            extract(b)

            @pl.when(i + 2 < n_chunks)
            def _():
                idx_cp(i + 2, b).start()

            store_cp(i, b).start()

        def body(g, carry):
            chunk_step(2 * g, 0)
            chunk_step(2 * g + 1, 1)
            return carry

        lax.fori_loop(0, n_chunks // 2, body, 0)
        store_cp(n_chunks - 2, 0).wait()
        store_cp(n_chunks - 1, 1).wait()

    return k


def kernel(sequence, table):
    bsz, hist = sequence.shape
    B = bsz * hist
    idx = sequence.reshape(B).astype(jnp.int32)
    table128 = table.reshape(table.shape[0] // 4, 4 * _D)
    out128 = _make_gather(B, 256)(table128, idx)
    return out128.reshape(bsz, hist, _D)


# speedup vs baseline: 2.5202x; 2.5202x over previous
import functools

import jax
import jax.numpy as jnp
from jax import lax
from jax.experimental import pallas as pl
from jax.experimental.pallas import tpu as pltpu
from jax.experimental.pallas import tpu_sc as plsc

_D = 32
_NC = 2
_NS = 16
_NW = _NC * _NS
_NB = 4


@functools.lru_cache(maxsize=None)
def _make_gather(BSZ, HIST, D, RPC):
    B = BSZ * HIST
    C = RPC * HIST
    b_per_w = B // _NW
    n_chunks = b_per_w // C
    assert n_chunks * C == b_per_w and n_chunks % _NB == 0, (B, C)
    mesh = plsc.VectorSubcoreMesh(core_axis_name="c", subcore_axis_name="s")

    @functools.partial(
        pl.kernel,
        mesh=mesh,
        out_type=jax.ShapeDtypeStruct((BSZ, HIST, D), jnp.float32),
        scratch_types=[
            pltpu.VMEM((_NB, C), jnp.int32),
            pltpu.VMEM((_NB, C, D), jnp.float32),
            pltpu.SemaphoreType.DMA((_NB,)),
            pltpu.SemaphoreType.DMA((_NB,)),
            pltpu.SemaphoreType.DMA((_NB,)),
        ],
        compiler_params=pltpu.CompilerParams(use_tc_tiling_on_sc=False),
    )
    def k(table_hbm, idx_hbm, out_hbm, idx_v, rows_v, sem_i, sem_g, sem_s):
        wid = lax.axis_index("s") * _NC + lax.axis_index("c")
        base = wid * b_per_w

        def idx_cp(i, b):
            return pltpu.make_async_copy(
                idx_hbm.at[pl.ds(base + i * C, C)], idx_v.at[b], sem_i.at[b])

        def gather_cp(b):
            return pltpu.make_async_copy(
                table_hbm.at[idx_v.at[b]], rows_v.at[b], sem_g.at[b])

        def store_cps(i, b):
            r0 = (base + i * C) // HIST
            return [
                pltpu.make_async_copy(
                    rows_v.at[b].at[pl.ds(r * HIST, HIST)],
                    out_hbm.at[r0 + r], sem_s.at[b])
                for r in range(RPC)
            ]

        for b in range(_NB):
            idx_cp(b, b).start()
        idx_cp(0, 0).wait()
        gather_cp(0).start()
        idx_cp(1, 1).wait()
        gather_cp(1).start()

        def chunk_step(i, b):
            gather_cp(b).wait()

            @pl.when(i + _NB < n_chunks)
            def _():
                idx_cp(i + _NB, b).start()

            @pl.when(i + 2 < n_chunks)
            def _():
                idx_cp(i + 2, (b + 2) % _NB).wait()

                @pl.when(i >= 2)
                def _():
                    for cp in store_cps(i - 2, (b + 2) % _NB):
                        cp.wait()
                gather_cp((b + 2) % _NB).start()

            for cp in store_cps(i, b):
                cp.start()

        def body(g, carry):
            for u in range(_NB):
                chunk_step(_NB * g + u, u)
            return carry

        lax.fori_loop(0, n_chunks // _NB, body, 0)
        for i in range(n_chunks - 2, n_chunks):
            for cp in store_cps(i, i % _NB):
                cp.wait()

    return k


def kernel(sequence, table):
    bsz, hist = sequence.shape
    B = bsz * hist
    idx = sequence.reshape(B).astype(jnp.int32)
    return _make_gather(bsz, hist, _D, 4)(table, idx)
